# Initial kernel scaffold; baseline (speedup 1.0000x reference)
import jax
import jax.numpy as jnp
from jax.experimental import pallas as pl


def kernel(inputs):
    raise NotImplementedError("write your pallas kernel here")


# speedup vs baseline: 1.3990x; 1.3990x over previous
import math

import jax
import jax.numpy as jnp
from jax.experimental import pallas as pl
from jax.experimental.pallas import tpu as pltpu

SIZE = 384
K = 20
B = 8
TWO_PI = 2.0 * math.pi

_INTERPRET = False


def _stage1_kernel(re_ref, im_ref, scores_ref, phase_ref):
    ampn = []
    res = []
    ims = []
    for c in range(3):
        re = re_ref[0, c]
        im = im_ref[0, c]
        ar = jnp.abs(re)
        ai = jnp.abs(im)
        mx = jnp.maximum(ar, ai)
        mn = jnp.minimum(ar, ai)
        t = mn / mx
        amp = jnp.where(mx == 0.0, 0.0, mx * jnp.sqrt(1.0 + t * t))
        ampn.append(amp / jnp.max(amp))
        res.append(re)
        ims.append(im)
    a0, a1, a2 = ampn
    c1gt = a1 > a0
    m01 = jnp.maximum(a0, a1)
    c2gt = a2 > m01
    re_sel = jnp.where(c2gt, res[2], jnp.where(c1gt, res[1], res[0]))
    im_sel = jnp.where(c2gt, ims[2], jnp.where(c1gt, ims[1], ims[0]))
    phase_ref[0] = jnp.arctan2(im_sel, re_sel) * (1.0 / math.pi)

    r = jax.lax.broadcasted_iota(jnp.int32, (SIZE, SIZE), 0)
    c = jax.lax.broadcasted_iota(jnp.int32, (SIZE, SIZE), 1)
    amp_max = jnp.maximum(m01, a2)
    amp_max = jnp.where((r + c) <= SIZE, amp_max, 0.0)
    zrow = jnp.zeros((1, SIZE), jnp.float32)
    zcol = jnp.zeros((SIZE, 1), jnp.float32)
    up = jnp.concatenate([amp_max[1:], zrow], axis=0)
    dn = jnp.concatenate([zrow, amp_max[:-1]], axis=0)
    vmax = jnp.maximum(amp_max, jnp.maximum(up, dn))
    lf = jnp.concatenate([vmax[:, 1:], zcol], axis=1)
    rt = jnp.concatenate([zcol, vmax[:, :-1]], axis=1)
    pooled = jnp.maximum(vmax, jnp.maximum(lf, rt))
    nms = jnp.where(amp_max < pooled, 0.0, amp_max)
    corner = ((r < 2) | (r >= SIZE - 2)) & ((c < 2) | (c >= SIZE - 2))
    scores_ref[0] = jnp.where(corner, 0.0, nms)


def _topk_kernel(scores_ref, phase_ref, idx_ref, tp_ref):
    rows, cols = scores_ref.shape[1], scores_ref.shape[2]
    r = jax.lax.broadcasted_iota(jnp.int32, (rows, cols), 0)
    c = jax.lax.broadcasted_iota(jnp.int32, (rows, cols), 1)
    lin = r * cols + c
    s = scores_ref[0]
    ph = phase_ref[0]
    for k in range(K):
        m = jnp.max(s)
        am = jnp.min(jnp.where(s == m, lin, jnp.int32(2 ** 30)))
        hit = lin == am
        tp = jnp.sum(jnp.where(hit, ph, 0.0))
        idx_ref[0, 0, k] = am
        tp_ref[0, 0, k] = tp
        s = jnp.where(hit, -1.0, s)


def _draw_kernel(idx_ref, tp_ref, out_ref):
    b = pl.program_id(0)
    k = pl.program_id(1)
    idx = idx_ref[b, k]
    u = idx // SIZE
    v = idx - u * SIZE
    tp = tp_ref[b, k]
    scale = jnp.float32(TWO_PI / SIZE)
    y = jax.lax.broadcasted_iota(jnp.int32, (SIZE, 1), 0)
    ay = tp * jnp.float32(math.pi) + ((u * y) % SIZE).astype(jnp.float32) * scale
    x = jax.lax.broadcasted_iota(jnp.int32, (1, SIZE), 1)
    bx = ((v * x) % SIZE).astype(jnp.float32) * scale
    out_ref[0, 0] = jnp.cos(ay) * jnp.cos(bx) - jnp.sin(ay) * jnp.sin(bx)


@jax.jit
def kernel(inputs):
    f = jnp.fft.fft2(inputs, axes=(1, 2))
    re = jnp.transpose(jnp.real(f), (0, 3, 1, 2))
    im = jnp.transpose(jnp.imag(f), (0, 3, 1, 2))
    scores, phase = pl.pallas_call(
        _stage1_kernel,
        grid=(B,),
        in_specs=[
            pl.BlockSpec((1, 3, SIZE, SIZE), lambda b: (b, 0, 0, 0)),
            pl.BlockSpec((1, 3, SIZE, SIZE), lambda b: (b, 0, 0, 0)),
        ],
        out_specs=[
            pl.BlockSpec((1, SIZE, SIZE), lambda b: (b, 0, 0)),
            pl.BlockSpec((1, SIZE, SIZE), lambda b: (b, 0, 0)),
        ],
        out_shape=[
            jax.ShapeDtypeStruct((B, SIZE, SIZE), jnp.float32),
            jax.ShapeDtypeStruct((B, SIZE, SIZE), jnp.float32),
        ],
        interpret=_INTERPRET,
    )(re, im)

    rows = SIZE * SIZE // 128
    s3 = scores.reshape(B, rows, 128)
    p3 = phase.reshape(B, rows, 128)
    idx, tp = pl.pallas_call(
        _topk_kernel,
        grid=(B,),
        in_specs=[
            pl.BlockSpec((1, rows, 128), lambda b: (b, 0, 0)),
            pl.BlockSpec((1, rows, 128), lambda b: (b, 0, 0)),
        ],
        out_specs=[
            pl.BlockSpec((1, 1, K), lambda b: (b, 0, 0), memory_space=pltpu.SMEM),
            pl.BlockSpec((1, 1, K), lambda b: (b, 0, 0), memory_space=pltpu.SMEM),
        ],
        out_shape=[
            jax.ShapeDtypeStruct((B, 1, K), jnp.int32),
            jax.ShapeDtypeStruct((B, 1, K), jnp.float32),
        ],
        interpret=_INTERPRET,
    )(s3, p3)
    idx = idx.reshape(B, K)
    tp = tp.reshape(B, K)

    iuvx = pl.pallas_call(
        _draw_kernel,
        grid=(B, K),
        in_specs=[
            pl.BlockSpec(memory_space=pltpu.SMEM),
            pl.BlockSpec(memory_space=pltpu.SMEM),
        ],
        out_specs=pl.BlockSpec((1, 1, SIZE, SIZE), lambda b, k: (b, k, 0, 0)),
        out_shape=jax.ShapeDtypeStruct((B, K, SIZE, SIZE), jnp.float32),
        interpret=_INTERPRET,
    )(idx, tp)

    u = idx // SIZE
    v = idx % SIZE
    coords = jnp.stack([u, v], axis=-1)
    top_coords = jnp.where(coords > SIZE // 2, coords - SIZE, coords)
    top_phases = tp[..., None]
    return (top_phases, top_coords, iuvx)


# speedup vs baseline: 1.4680x; 1.0494x over previous
import math

import jax
import jax.numpy as jnp
from jax.experimental import pallas as pl
from jax.experimental.pallas import tpu as pltpu

SIZE = 384
K = 20
B = 8
TWO_PI = 2.0 * math.pi

_INTERPRET = False


def _stage1_kernel(re_ref, im_ref, scores_ref, phase_ref):
    ampn = []
    res = []
    ims = []
    for c in range(3):
        re = re_ref[0, c]
        im = im_ref[0, c]
        ar = jnp.abs(re)
        ai = jnp.abs(im)
        mx = jnp.maximum(ar, ai)
        mn = jnp.minimum(ar, ai)
        t = mn / mx
        amp = jnp.where(mx == 0.0, 0.0, mx * jnp.sqrt(1.0 + t * t))
        ampn.append(amp / jnp.max(amp))
        res.append(re)
        ims.append(im)
    a0, a1, a2 = ampn
    c1gt = a1 > a0
    m01 = jnp.maximum(a0, a1)
    c2gt = a2 > m01
    re_sel = jnp.where(c2gt, res[2], jnp.where(c1gt, res[1], res[0]))
    im_sel = jnp.where(c2gt, ims[2], jnp.where(c1gt, ims[1], ims[0]))
    phase_ref[0] = jnp.arctan2(im_sel, re_sel) * (1.0 / math.pi)

    r = jax.lax.broadcasted_iota(jnp.int32, (SIZE, SIZE), 0)
    c = jax.lax.broadcasted_iota(jnp.int32, (SIZE, SIZE), 1)
    amp_max = jnp.maximum(m01, a2)
    amp_max = jnp.where((r + c) <= SIZE, amp_max, 0.0)
    zrow = jnp.zeros((1, SIZE), jnp.float32)
    zcol = jnp.zeros((SIZE, 1), jnp.float32)
    up = jnp.concatenate([amp_max[1:], zrow], axis=0)
    dn = jnp.concatenate([zrow, amp_max[:-1]], axis=0)
    vmax = jnp.maximum(amp_max, jnp.maximum(up, dn))
    lf = jnp.concatenate([vmax[:, 1:], zcol], axis=1)
    rt = jnp.concatenate([zcol, vmax[:, :-1]], axis=1)
    pooled = jnp.maximum(vmax, jnp.maximum(lf, rt))
    nms = jnp.where(amp_max < pooled, 0.0, amp_max)
    corner = ((r < 2) | (r >= SIZE - 2)) & ((c < 2) | (c >= SIZE - 2))
    scores_ref[0] = jnp.where(corner, 0.0, nms)


def _topk_kernel(scores_ref, phase_ref, idx_ref, tp_ref, s_scratch):
    nslab = 9
    s_scratch[...] = scores_ref[0]
    m_acc = s_scratch[0:128, :]
    j_acc = jnp.zeros((128, 128), jnp.int32)
    for j in range(1, nslab):
        slab = s_scratch[j * 128:(j + 1) * 128, :]
        upd = slab > m_acc
        j_acc = jnp.where(upd, j, j_acc)
        m_acc = jnp.maximum(m_acc, slab)
    r128 = jax.lax.broadcasted_iota(jnp.int32, (128, 128), 0)
    lane = jax.lax.broadcasted_iota(jnp.int32, (128, 128), 1)
    lane_row = jax.lax.broadcasted_iota(jnp.int32, (1, 128), 1)
    lin_full = j_acc * (128 * 128) + r128 * 128 + lane
    for k in range(K):
        m = jnp.max(m_acc)
        am = jnp.min(jnp.where(m_acc == m, lin_full, jnp.int32(2 ** 30)))
        row = am // 128
        ln = am - row * 128
        ph_row = phase_ref[0, pl.ds(row, 1), :]
        tp = jnp.sum(jnp.where(lane_row == ln, ph_row, 0.0))
        idx_ref[0, 0, k] = am
        tp_ref[0, 0, k] = tp
        cell = row % 128
        win_row = s_scratch[pl.ds(row, 1), :]
        s_scratch[pl.ds(row, 1), :] = jnp.where(lane_row == ln, -1.0, win_row)
        new_m = s_scratch[pl.ds(cell, 1), :]
        new_j = jnp.zeros((1, 128), jnp.int32)
        for j in range(1, nslab):
            slabr = s_scratch[pl.ds(j * 128 + cell, 1), :]
            upd = slabr > new_m
            new_j = jnp.where(upd, j, new_j)
            new_m = jnp.maximum(new_m, slabr)
        keep = jax.lax.broadcasted_iota(jnp.int32, (128, 128), 0) == cell
        m_acc = jnp.where(keep, new_m, m_acc)
        j_acc = jnp.where(keep, new_j, j_acc)
        lin_full = j_acc * (128 * 128) + r128 * 128 + lane


KCHUNK = 5


def _draw_kernel(idx_ref, tp_ref, out_ref):
    b = pl.program_id(0)
    kc = pl.program_id(1)
    scale = jnp.float32(TWO_PI / SIZE)
    y = jax.lax.broadcasted_iota(jnp.int32, (SIZE, 1), 0)
    x = jax.lax.broadcasted_iota(jnp.int32, (1, SIZE), 1)
    for kk in range(KCHUNK):
        k = kc * KCHUNK + kk
        idx = idx_ref[b, k]
        u = idx // SIZE
        v = idx - u * SIZE
        tp = tp_ref[b, k]
        ay = tp * jnp.float32(math.pi) + ((u * y) % SIZE).astype(jnp.float32) * scale
        bx = ((v * x) % SIZE).astype(jnp.float32) * scale
        out_ref[0, kk] = jnp.cos(ay) * jnp.cos(bx) - jnp.sin(ay) * jnp.sin(bx)


@jax.jit
def kernel(inputs):
    f = jnp.fft.fft2(inputs, axes=(1, 2))
    re = jnp.transpose(jnp.real(f), (0, 3, 1, 2))
    im = jnp.transpose(jnp.imag(f), (0, 3, 1, 2))
    scores, phase = pl.pallas_call(
        _stage1_kernel,
        grid=(B,),
        in_specs=[
            pl.BlockSpec((1, 3, SIZE, SIZE), lambda b: (b, 0, 0, 0)),
            pl.BlockSpec((1, 3, SIZE, SIZE), lambda b: (b, 0, 0, 0)),
        ],
        out_specs=[
            pl.BlockSpec((1, SIZE, SIZE), lambda b: (b, 0, 0)),
            pl.BlockSpec((1, SIZE, SIZE), lambda b: (b, 0, 0)),
        ],
        out_shape=[
            jax.ShapeDtypeStruct((B, SIZE, SIZE), jnp.float32),
            jax.ShapeDtypeStruct((B, SIZE, SIZE), jnp.float32),
        ],
        interpret=_INTERPRET,
    )(re, im)

    rows = SIZE * SIZE // 128
    s3 = scores.reshape(B, rows, 128)
    p3 = phase.reshape(B, rows, 128)
    idx, tp = pl.pallas_call(
        _topk_kernel,
        grid=(B,),
        in_specs=[
            pl.BlockSpec((1, rows, 128), lambda b: (b, 0, 0)),
            pl.BlockSpec((1, rows, 128), lambda b: (b, 0, 0)),
        ],
        out_specs=[
            pl.BlockSpec((1, 1, K), lambda b: (b, 0, 0), memory_space=pltpu.SMEM),
            pl.BlockSpec((1, 1, K), lambda b: (b, 0, 0), memory_space=pltpu.SMEM),
        ],
        out_shape=[
            jax.ShapeDtypeStruct((B, 1, K), jnp.int32),
            jax.ShapeDtypeStruct((B, 1, K), jnp.float32),
        ],
        scratch_shapes=[pltpu.VMEM((rows, 128), jnp.float32)],
        interpret=_INTERPRET,
    )(s3, p3)
    idx = idx.reshape(B, K)
    tp = tp.reshape(B, K)

    iuvx = pl.pallas_call(
        _draw_kernel,
        grid=(B, K // KCHUNK),
        in_specs=[
            pl.BlockSpec(memory_space=pltpu.SMEM),
            pl.BlockSpec(memory_space=pltpu.SMEM),
        ],
        out_specs=pl.BlockSpec((1, KCHUNK, SIZE, SIZE),
                               lambda b, k: (b, k, 0, 0)),
        out_shape=jax.ShapeDtypeStruct((B, K, SIZE, SIZE), jnp.float32),
        interpret=_INTERPRET,
    )(idx, tp)

    u = idx // SIZE
    v = idx % SIZE
    coords = jnp.stack([u, v], axis=-1)
    top_coords = jnp.where(coords > SIZE // 2, coords - SIZE, coords)
    top_phases = tp[..., None]
    return (top_phases, top_coords, iuvx)


# speedup vs baseline: 1.5759x; 1.0735x over previous
import math

import jax
import jax.numpy as jnp
from jax.experimental import pallas as pl
from jax.experimental.pallas import tpu as pltpu

SIZE = 384
K = 20
B = 8
TWO_PI = 2.0 * math.pi

_INTERPRET = False


def _stage1_kernel(re_ref, im_ref, scores_ref, phase_ref):
    ampn = []
    res = []
    ims = []
    for c in range(3):
        re = re_ref[0, c]
        im = im_ref[0, c]
        ar = jnp.abs(re)
        ai = jnp.abs(im)
        mx = jnp.maximum(ar, ai)
        mn = jnp.minimum(ar, ai)
        t = mn / mx
        amp = jnp.where(mx == 0.0, 0.0, mx * jnp.sqrt(1.0 + t * t))
        ampn.append(amp / jnp.max(amp))
        res.append(re)
        ims.append(im)
    a0, a1, a2 = ampn
    c1gt = a1 > a0
    m01 = jnp.maximum(a0, a1)
    c2gt = a2 > m01
    re_sel = jnp.where(c2gt, res[2], jnp.where(c1gt, res[1], res[0]))
    im_sel = jnp.where(c2gt, ims[2], jnp.where(c1gt, ims[1], ims[0]))
    phase_ref[0] = jnp.arctan2(im_sel, re_sel) * (1.0 / math.pi)

    r = jax.lax.broadcasted_iota(jnp.int32, (SIZE, SIZE), 0)
    c = jax.lax.broadcasted_iota(jnp.int32, (SIZE, SIZE), 1)
    amp_max = jnp.maximum(m01, a2)
    amp_max = jnp.where((r + c) <= SIZE, amp_max, 0.0)
    zrow = jnp.zeros((1, SIZE), jnp.float32)
    zcol = jnp.zeros((SIZE, 1), jnp.float32)
    up = jnp.concatenate([amp_max[1:], zrow], axis=0)
    dn = jnp.concatenate([zrow, amp_max[:-1]], axis=0)
    vmax = jnp.maximum(amp_max, jnp.maximum(up, dn))
    lf = jnp.concatenate([vmax[:, 1:], zcol], axis=1)
    rt = jnp.concatenate([zcol, vmax[:, :-1]], axis=1)
    pooled = jnp.maximum(vmax, jnp.maximum(lf, rt))
    nms = jnp.where(amp_max < pooled, 0.0, amp_max)
    corner = ((r < 2) | (r >= SIZE - 2)) & ((c < 2) | (c >= SIZE - 2))
    scores_ref[0] = jnp.where(corner, 0.0, nms)


def _topk_kernel(scores_ref, phase_ref, idx_ref, tp_ref, s_scratch):
    nslab = 9
    s_scratch[...] = scores_ref[...]
    r128 = jax.lax.broadcasted_iota(jnp.int32, (128, 128), 0)
    lane = jax.lax.broadcasted_iota(jnp.int32, (128, 128), 1)
    lane_row = jax.lax.broadcasted_iota(jnp.int32, (1, 128), 1)
    lin_base = r128 * 128 + lane
    m_accs = [None] * B
    j_accs = [None] * B
    for b in range(B):
        m_acc = s_scratch[b, 0:128, :]
        j_acc = jnp.zeros((128, 128), jnp.int32)
        for j in range(1, nslab):
            slab = s_scratch[b, j * 128:(j + 1) * 128, :]
            upd = slab > m_acc
            j_acc = jnp.where(upd, j, j_acc)
            m_acc = jnp.maximum(m_acc, slab)
        m_accs[b] = m_acc
        j_accs[b] = j_acc
    for k in range(K):
        for b in range(B):
            m_acc = m_accs[b]
            j_acc = j_accs[b]
            m = jnp.max(m_acc)
            am = jnp.min(jnp.where(
                m_acc == m, j_acc * (128 * 128) + lin_base, jnp.int32(2 ** 30)))
            row = am // 128
            ln = am - row * 128
            ph_row = phase_ref[b, pl.ds(row, 1), :]
            tp = jnp.sum(jnp.where(lane_row == ln, ph_row, 0.0))
            idx_ref[b, 0, k] = am
            tp_ref[b, 0, k] = tp
            cell = row % 128
            win_row = s_scratch[b, pl.ds(row, 1), :]
            s_scratch[b, pl.ds(row, 1), :] = jnp.where(lane_row == ln, -1.0, win_row)
            new_m = s_scratch[b, pl.ds(cell, 1), :]
            new_j = jnp.zeros((1, 128), jnp.int32)
            for j in range(1, nslab):
                slabr = s_scratch[b, pl.ds(j * 128 + cell, 1), :]
                upd = slabr > new_m
                new_j = jnp.where(upd, j, new_j)
                new_m = jnp.maximum(new_m, slabr)
            keep = r128 == cell
            m_accs[b] = jnp.where(keep, new_m, m_acc)
            j_accs[b] = jnp.where(keep, new_j, j_acc)


KCHUNK = 5


def _draw_kernel(idx_ref, tp_ref, out_ref):
    b = pl.program_id(0)
    kc = pl.program_id(1)
    scale = jnp.float32(TWO_PI / SIZE)
    x = jax.lax.broadcasted_iota(jnp.int32, (1, SIZE), 1)
    dims = (((0,), (0,)), ((), ()))
    for kk in range(KCHUNK):
        k = kc * KCHUNK + kk
        idx = idx_ref[b, k]
        u = idx // SIZE
        v = idx - u * SIZE
        tp = tp_ref[b, k]
        ay = tp * jnp.float32(math.pi) + ((u * x) % SIZE).astype(jnp.float32) * scale
        bx = ((v * x) % SIZE).astype(jnp.float32) * scale
        a = jnp.concatenate([jnp.cos(ay), jnp.sin(ay)], axis=0)
        bm = jnp.concatenate([jnp.cos(bx), -jnp.sin(bx)], axis=0)
        out_ref[0, kk] = jax.lax.dot_general(
            a, bm, dims, precision=jax.lax.Precision.HIGHEST,
            preferred_element_type=jnp.float32)


@jax.jit
def kernel(inputs):
    f = jnp.fft.fft2(inputs, axes=(1, 2))
    re = jnp.transpose(jnp.real(f), (0, 3, 1, 2))
    im = jnp.transpose(jnp.imag(f), (0, 3, 1, 2))
    scores, phase = pl.pallas_call(
        _stage1_kernel,
        grid=(B,),
        in_specs=[
            pl.BlockSpec((1, 3, SIZE, SIZE), lambda b: (b, 0, 0, 0)),
            pl.BlockSpec((1, 3, SIZE, SIZE), lambda b: (b, 0, 0, 0)),
        ],
        out_specs=[
            pl.BlockSpec((1, SIZE, SIZE), lambda b: (b, 0, 0)),
            pl.BlockSpec((1, SIZE, SIZE), lambda b: (b, 0, 0)),
        ],
        out_shape=[
            jax.ShapeDtypeStruct((B, SIZE, SIZE), jnp.float32),
            jax.ShapeDtypeStruct((B, SIZE, SIZE), jnp.float32),
        ],
        interpret=_INTERPRET,
    )(re, im)

    rows = SIZE * SIZE // 128
    s3 = scores.reshape(B, rows, 128)
    p3 = phase.reshape(B, rows, 128)
    idx, tp = pl.pallas_call(
        _topk_kernel,
        grid=(1,),
        in_specs=[
            pl.BlockSpec((B, rows, 128), lambda b: (0, 0, 0)),
            pl.BlockSpec((B, rows, 128), lambda b: (0, 0, 0)),
        ],
        out_specs=[
            pl.BlockSpec((B, 1, K), lambda b: (0, 0, 0), memory_space=pltpu.SMEM),
            pl.BlockSpec((B, 1, K), lambda b: (0, 0, 0), memory_space=pltpu.SMEM),
        ],
        out_shape=[
            jax.ShapeDtypeStruct((B, 1, K), jnp.int32),
            jax.ShapeDtypeStruct((B, 1, K), jnp.float32),
        ],
        scratch_shapes=[pltpu.VMEM((B, rows, 128), jnp.float32)],
        interpret=_INTERPRET,
    )(s3, p3)
    idx = idx.reshape(B, K)
    tp = tp.reshape(B, K)

    iuvx = pl.pallas_call(
        _draw_kernel,
        grid=(B, K // KCHUNK),
        in_specs=[
            pl.BlockSpec(memory_space=pltpu.SMEM),
            pl.BlockSpec(memory_space=pltpu.SMEM),
        ],
        out_specs=pl.BlockSpec((1, KCHUNK, SIZE, SIZE),
                               lambda b, k: (b, k, 0, 0)),
        out_shape=jax.ShapeDtypeStruct((B, K, SIZE, SIZE), jnp.float32),
        interpret=_INTERPRET,
    )(idx, tp)

    u = idx // SIZE
    v = idx % SIZE
    coords = jnp.stack([u, v], axis=-1)
    top_coords = jnp.where(coords > SIZE // 2, coords - SIZE, coords)
    top_phases = tp[..., None]
    return (top_phases, top_coords, iuvx)


# speedup vs baseline: 1.5848x; 1.0057x over previous
import math

import jax
import jax.numpy as jnp
from jax.experimental import pallas as pl
from jax.experimental.pallas import tpu as pltpu

SIZE = 384
K = 20
B = 8
TWO_PI = 2.0 * math.pi

_INTERPRET = False


def _stage1_kernel(re_ref, im_ref, scores_ref, phase_ref):
    ampn = []
    res = []
    ims = []
    for c in range(3):
        re = re_ref[0, c]
        im = im_ref[0, c]
        ar = jnp.abs(re)
        ai = jnp.abs(im)
        mx = jnp.maximum(ar, ai)
        mn = jnp.minimum(ar, ai)
        t = mn / mx
        amp = jnp.where(mx == 0.0, 0.0, mx * jnp.sqrt(1.0 + t * t))
        ampn.append(amp / jnp.max(amp))
        res.append(re)
        ims.append(im)
    a0, a1, a2 = ampn
    c1gt = a1 > a0
    m01 = jnp.maximum(a0, a1)
    c2gt = a2 > m01
    re_sel = jnp.where(c2gt, res[2], jnp.where(c1gt, res[1], res[0]))
    im_sel = jnp.where(c2gt, ims[2], jnp.where(c1gt, ims[1], ims[0]))
    phase_ref[0] = jnp.arctan2(im_sel, re_sel) * (1.0 / math.pi)

    r = jax.lax.broadcasted_iota(jnp.int32, (SIZE, SIZE), 0)
    c = jax.lax.broadcasted_iota(jnp.int32, (SIZE, SIZE), 1)
    amp_max = jnp.maximum(m01, a2)
    amp_max = jnp.where((r + c) <= SIZE, amp_max, 0.0)
    zrow = jnp.zeros((1, SIZE), jnp.float32)
    zcol = jnp.zeros((SIZE, 1), jnp.float32)
    up = jnp.concatenate([amp_max[1:], zrow], axis=0)
    dn = jnp.concatenate([zrow, amp_max[:-1]], axis=0)
    vmax = jnp.maximum(amp_max, jnp.maximum(up, dn))
    lf = jnp.concatenate([vmax[:, 1:], zcol], axis=1)
    rt = jnp.concatenate([zcol, vmax[:, :-1]], axis=1)
    pooled = jnp.maximum(vmax, jnp.maximum(lf, rt))
    nms = jnp.where(amp_max < pooled, 0.0, amp_max)
    corner = ((r < 2) | (r >= SIZE - 2)) & ((c < 2) | (c >= SIZE - 2))
    scores_ref[0] = jnp.where(corner, 0.0, nms)


def _topk_kernel(scores_ref, phase_ref, idx_ref, tp_ref, s_scratch):
    nslab = 9
    s_scratch[...] = scores_ref[...]
    r128 = jax.lax.broadcasted_iota(jnp.int32, (128, 128), 0)
    lane = jax.lax.broadcasted_iota(jnp.int32, (128, 128), 1)
    lane_row = jax.lax.broadcasted_iota(jnp.int32, (1, 128), 1)
    lin_base = r128 * 128 + lane
    m_accs = [None] * B
    j_accs = [None] * B
    for b in range(B):
        m_acc = s_scratch[b, 0:128, :]
        j_acc = jnp.zeros((128, 128), jnp.int32)
        for j in range(1, nslab):
            slab = s_scratch[b, j * 128:(j + 1) * 128, :]
            upd = slab > m_acc
            j_acc = jnp.where(upd, j, j_acc)
            m_acc = jnp.maximum(m_acc, slab)
        m_accs[b] = m_acc
        j_accs[b] = j_acc
    for k in range(K):
        for b in range(B):
            m_acc = m_accs[b]
            j_acc = j_accs[b]
            m = jnp.max(m_acc)
            am = jnp.min(jnp.where(
                m_acc == m, j_acc * (128 * 128) + lin_base, jnp.int32(2 ** 30)))
            row = am // 128
            ln = am - row * 128
            ph_row = phase_ref[b, pl.ds(row, 1), :]
            tp = jnp.sum(jnp.where(lane_row == ln, ph_row, 0.0))
            idx_ref[b, 0, k] = am
            tp_ref[b, 0, k] = tp
            cell = row % 128
            win_row = s_scratch[b, pl.ds(row, 1), :]
            s_scratch[b, pl.ds(row, 1), :] = jnp.where(lane_row == ln, -1.0, win_row)
            new_m = s_scratch[b, pl.ds(cell, 1), :]
            new_j = jnp.zeros((1, 128), jnp.int32)
            for j in range(1, nslab):
                slabr = s_scratch[b, pl.ds(j * 128 + cell, 1), :]
                upd = slabr > new_m
                new_j = jnp.where(upd, j, new_j)
                new_m = jnp.maximum(new_m, slabr)
            keep = r128 == cell
            m_accs[b] = jnp.where(keep, new_m, m_acc)
            j_accs[b] = jnp.where(keep, new_j, j_acc)


KCHUNK = 10


def _draw_kernel(idx_ref, tp_ref, out_ref):
    b = pl.program_id(0)
    kc = pl.program_id(1)
    scale = jnp.float32(TWO_PI / SIZE)
    x = jax.lax.broadcasted_iota(jnp.int32, (1, SIZE), 1)
    dims = (((0,), (0,)), ((), ()))
    for kk in range(KCHUNK):
        k = kc * KCHUNK + kk
        idx = idx_ref[b, k]
        u = idx // SIZE
        v = idx - u * SIZE
        tp = tp_ref[b, k]
        ay = tp * jnp.float32(math.pi) + ((u * x) % SIZE).astype(jnp.float32) * scale
        bx = ((v * x) % SIZE).astype(jnp.float32) * scale
        a = jnp.concatenate([jnp.cos(ay), jnp.sin(ay)], axis=0)
        bm = jnp.concatenate([jnp.cos(bx), -jnp.sin(bx)], axis=0)
        out_ref[0, kk] = jax.lax.dot_general(
            a, bm, dims, precision=jax.lax.Precision.HIGHEST,
            preferred_element_type=jnp.float32)


@jax.jit
def kernel(inputs):
    f = jnp.fft.fft2(inputs, axes=(1, 2))
    re = jnp.transpose(jnp.real(f), (0, 3, 1, 2))
    im = jnp.transpose(jnp.imag(f), (0, 3, 1, 2))
    scores, phase = pl.pallas_call(
        _stage1_kernel,
        grid=(B,),
        in_specs=[
            pl.BlockSpec((1, 3, SIZE, SIZE), lambda b: (b, 0, 0, 0)),
            pl.BlockSpec((1, 3, SIZE, SIZE), lambda b: (b, 0, 0, 0)),
        ],
        out_specs=[
            pl.BlockSpec((1, SIZE, SIZE), lambda b: (b, 0, 0)),
            pl.BlockSpec((1, SIZE, SIZE), lambda b: (b, 0, 0)),
        ],
        out_shape=[
            jax.ShapeDtypeStruct((B, SIZE, SIZE), jnp.float32),
            jax.ShapeDtypeStruct((B, SIZE, SIZE), jnp.float32),
        ],
        interpret=_INTERPRET,
    )(re, im)

    rows = SIZE * SIZE // 128
    s3 = scores.reshape(B, rows, 128)
    p3 = phase.reshape(B, rows, 128)
    idx, tp = pl.pallas_call(
        _topk_kernel,
        grid=(1,),
        in_specs=[
            pl.BlockSpec((B, rows, 128), lambda b: (0, 0, 0)),
            pl.BlockSpec((B, rows, 128), lambda b: (0, 0, 0)),
        ],
        out_specs=[
            pl.BlockSpec((B, 1, K), lambda b: (0, 0, 0), memory_space=pltpu.SMEM),
            pl.BlockSpec((B, 1, K), lambda b: (0, 0, 0), memory_space=pltpu.SMEM),
        ],
        out_shape=[
            jax.ShapeDtypeStruct((B, 1, K), jnp.int32),
            jax.ShapeDtypeStruct((B, 1, K), jnp.float32),
        ],
        scratch_shapes=[pltpu.VMEM((B, rows, 128), jnp.float32)],
        interpret=_INTERPRET,
    )(s3, p3)
    idx = idx.reshape(B, K)
    tp = tp.reshape(B, K)

    iuvx = pl.pallas_call(
        _draw_kernel,
        grid=(B, K // KCHUNK),
        in_specs=[
            pl.BlockSpec(memory_space=pltpu.SMEM),
            pl.BlockSpec(memory_space=pltpu.SMEM),
        ],
        out_specs=pl.BlockSpec((1, KCHUNK, SIZE, SIZE),
                               lambda b, k: (b, k, 0, 0)),
        out_shape=jax.ShapeDtypeStruct((B, K, SIZE, SIZE), jnp.float32),
        interpret=_INTERPRET,
    )(idx, tp)

    u = idx // SIZE
    v = idx % SIZE
    coords = jnp.stack([u, v], axis=-1)
    top_coords = jnp.where(coords > SIZE // 2, coords - SIZE, coords)
    top_phases = tp[..., None]
    return (top_phases, top_coords, iuvx)


# speedup vs baseline: 1.5863x; 1.0009x over previous
import math

import jax
import jax.numpy as jnp
from jax.experimental import pallas as pl
from jax.experimental.pallas import tpu as pltpu

SIZE = 384
K = 20
B = 8
TWO_PI = 2.0 * math.pi


def _stage1_kernel(re_ref, im_ref, scores_ref, phase_ref):
    ampn = []
    res = []
    ims = []
    for c in range(3):
        re = re_ref[0, c]
        im = im_ref[0, c]
        ar = jnp.abs(re)
        ai = jnp.abs(im)
        mx = jnp.maximum(ar, ai)
        mn = jnp.minimum(ar, ai)
        t = mn / mx
        amp = jnp.where(mx == 0.0, 0.0, mx * jnp.sqrt(1.0 + t * t))
        ampn.append(amp / jnp.max(amp))
        res.append(re)
        ims.append(im)
    a0, a1, a2 = ampn
    c1gt = a1 > a0
    m01 = jnp.maximum(a0, a1)
    c2gt = a2 > m01
    re_sel = jnp.where(c2gt, res[2], jnp.where(c1gt, res[1], res[0]))
    im_sel = jnp.where(c2gt, ims[2], jnp.where(c1gt, ims[1], ims[0]))
    phase_ref[0] = jnp.arctan2(im_sel, re_sel) * (1.0 / math.pi)

    r = jax.lax.broadcasted_iota(jnp.int32, (SIZE, SIZE), 0)
    c = jax.lax.broadcasted_iota(jnp.int32, (SIZE, SIZE), 1)
    amp_max = jnp.maximum(m01, a2)
    amp_max = jnp.where((r + c) <= SIZE, amp_max, 0.0)
    zrow = jnp.zeros((1, SIZE), jnp.float32)
    zcol = jnp.zeros((SIZE, 1), jnp.float32)
    up = jnp.concatenate([amp_max[1:], zrow], axis=0)
    dn = jnp.concatenate([zrow, amp_max[:-1]], axis=0)
    vmax = jnp.maximum(amp_max, jnp.maximum(up, dn))
    lf = jnp.concatenate([vmax[:, 1:], zcol], axis=1)
    rt = jnp.concatenate([zcol, vmax[:, :-1]], axis=1)
    pooled = jnp.maximum(vmax, jnp.maximum(lf, rt))
    nms = jnp.where(amp_max < pooled, 0.0, amp_max)
    corner = ((r < 2) | (r >= SIZE - 2)) & ((c < 2) | (c >= SIZE - 2))
    scores_ref[0] = jnp.where(corner, 0.0, nms)


def _topk_kernel(scores_ref, phase_ref, idx_ref, tp_ref, s_scratch):
    nslab = 9
    s_scratch[...] = scores_ref[...]
    r128 = jax.lax.broadcasted_iota(jnp.int32, (128, 128), 0)
    lane = jax.lax.broadcasted_iota(jnp.int32, (128, 128), 1)
    lane_row = jax.lax.broadcasted_iota(jnp.int32, (1, 128), 1)
    lin_base = r128 * 128 + lane
    m_accs = [None] * B
    j_accs = [None] * B
    for b in range(B):
        m_acc = s_scratch[b, 0:128, :]
        j_acc = jnp.zeros((128, 128), jnp.int32)
        for j in range(1, nslab):
            slab = s_scratch[b, j * 128:(j + 1) * 128, :]
            upd = slab > m_acc
            j_acc = jnp.where(upd, j, j_acc)
            m_acc = jnp.maximum(m_acc, slab)
        m_accs[b] = m_acc
        j_accs[b] = j_acc
    for k in range(K):
        for b in range(B):
            m_acc = m_accs[b]
            j_acc = j_accs[b]
            m = jnp.max(m_acc)
            am = jnp.min(jnp.where(
                m_acc == m, j_acc * (128 * 128) + lin_base, jnp.int32(2 ** 30)))
            row = am // 128
            ln = am - row * 128
            ph_row = phase_ref[b, pl.ds(row, 1), :]
            tp = jnp.sum(jnp.where(lane_row == ln, ph_row, 0.0))
            idx_ref[b, 0, k] = am
            tp_ref[b, 0, k] = tp
            cell = row % 128
            win_row = s_scratch[b, pl.ds(row, 1), :]
            s_scratch[b, pl.ds(row, 1), :] = jnp.where(lane_row == ln, -1.0, win_row)
            new_m = s_scratch[b, pl.ds(cell, 1), :]
            new_j = jnp.zeros((1, 128), jnp.int32)
            for j in range(1, nslab):
                slabr = s_scratch[b, pl.ds(j * 128 + cell, 1), :]
                upd = slabr > new_m
                new_j = jnp.where(upd, j, new_j)
                new_m = jnp.maximum(new_m, slabr)
            keep = r128 == cell
            m_accs[b] = jnp.where(keep, new_m, m_acc)
            j_accs[b] = jnp.where(keep, new_j, j_acc)


KCHUNK = 10


def _draw_kernel(idx_ref, tp_ref, out_ref):
    b = pl.program_id(0)
    kc = pl.program_id(1)
    scale = jnp.float32(TWO_PI / SIZE)
    x = jax.lax.broadcasted_iota(jnp.int32, (1, SIZE), 1)
    dims = (((0,), (0,)), ((), ()))
    for kk in range(KCHUNK):
        k = kc * KCHUNK + kk
        idx = idx_ref[b, k]
        u = idx // SIZE
        v = idx - u * SIZE
        tp = tp_ref[b, k]
        ay = tp * jnp.float32(math.pi) + ((u * x) % SIZE).astype(jnp.float32) * scale
        bx = ((v * x) % SIZE).astype(jnp.float32) * scale
        a = jnp.concatenate([jnp.cos(ay), jnp.sin(ay)], axis=0)
        bm = jnp.concatenate([jnp.cos(bx), -jnp.sin(bx)], axis=0)
        out_ref[0, kk] = jax.lax.dot_general(
            a, bm, dims, precision=jax.lax.Precision.HIGHEST,
            preferred_element_type=jnp.float32)


@jax.jit
def kernel(inputs):
    f = jnp.fft.fft2(inputs, axes=(1, 2))
    re = jnp.transpose(jnp.real(f), (0, 3, 1, 2))
    im = jnp.transpose(jnp.imag(f), (0, 3, 1, 2))
    scores, phase = pl.pallas_call(
        _stage1_kernel,
        grid=(B,),
        in_specs=[
            pl.BlockSpec((1, 3, SIZE, SIZE), lambda b: (b, 0, 0, 0)),
            pl.BlockSpec((1, 3, SIZE, SIZE), lambda b: (b, 0, 0, 0)),
        ],
        out_specs=[
            pl.BlockSpec((1, SIZE, SIZE), lambda b: (b, 0, 0)),
            pl.BlockSpec((1, SIZE, SIZE), lambda b: (b, 0, 0)),
        ],
        out_shape=[
            jax.ShapeDtypeStruct((B, SIZE, SIZE), jnp.float32),
            jax.ShapeDtypeStruct((B, SIZE, SIZE), jnp.float32),
        ],
    )(re, im)

    rows = SIZE * SIZE // 128
    s3 = scores.reshape(B, rows, 128)
    p3 = phase.reshape(B, rows, 128)
    idx, tp = pl.pallas_call(
        _topk_kernel,
        grid=(1,),
        in_specs=[
            pl.BlockSpec((B, rows, 128), lambda b: (0, 0, 0)),
            pl.BlockSpec((B, rows, 128), lambda b: (0, 0, 0)),
        ],
        out_specs=[
            pl.BlockSpec((B, 1, K), lambda b: (0, 0, 0), memory_space=pltpu.SMEM),
            pl.BlockSpec((B, 1, K), lambda b: (0, 0, 0), memory_space=pltpu.SMEM),
        ],
        out_shape=[
            jax.ShapeDtypeStruct((B, 1, K), jnp.int32),
            jax.ShapeDtypeStruct((B, 1, K), jnp.float32),
        ],
        scratch_shapes=[pltpu.VMEM((B, rows, 128), jnp.float32)],
    )(s3, p3)
    idx = idx.reshape(B, K)
    tp = tp.reshape(B, K)

    iuvx = pl.pallas_call(
        _draw_kernel,
        grid=(B, K // KCHUNK),
        in_specs=[
            pl.BlockSpec(memory_space=pltpu.SMEM),
            pl.BlockSpec(memory_space=pltpu.SMEM),
        ],
        out_specs=pl.BlockSpec((1, KCHUNK, SIZE, SIZE),
                               lambda b, k: (b, k, 0, 0)),
        out_shape=jax.ShapeDtypeStruct((B, K, SIZE, SIZE), jnp.float32),
    )(idx, tp)

    u = idx // SIZE
    v = idx % SIZE
    coords = jnp.stack([u, v], axis=-1)
    top_coords = jnp.where(coords > SIZE // 2, coords - SIZE, coords)
    top_phases = tp[..., None]
    return (top_phases, top_coords, iuvx)
